# baseline (speedup 1.0000x reference)
import functools

import jax
import jax.numpy as jnp
from jax import lax
from jax.experimental import pallas as pl
from jax.experimental.pallas import tpu as pltpu
from jax.experimental.pallas import tpu_sc as plsc

VOCAB = 30522
D = 768
B = 128
S = 512
EPS = 1e-12
NTOK = B * S
NC = 2
NS = 16
NW = NC * NS
TPW = NTOK // NW
CH = 16
NCHUNK = TPW // CH
DV = D // 16


_GATHER_DNUMS = lax.GatherDimensionNumbers(
    offset_dims=(), collapsed_slice_dims=(0,), start_index_map=(0,))


def _permute(x, idx):
    return lax.gather(x, idx[:, None], _GATHER_DNUMS, (1,),
                      mode=lax.GatherScatterMode.PROMISE_IN_BOUNDS)


def _lanesum(x):
    idx = lax.iota(jnp.int32, 16)
    for k in (1, 2, 4, 8):
        x = x + _permute(x, idx ^ k)
    return x


def _rsqrt_vec(x):
    xi = lax.bitcast_convert_type(x, jnp.int32)
    yi = jnp.int32(0x5F3759DF) - lax.shift_right_arithmetic(xi, 1)
    y = lax.bitcast_convert_type(yi, jnp.float32)
    for _ in range(3):
        y = y * (jnp.float32(1.5) - jnp.float32(0.5) * x * y * y)
    return y


def _sc_body(ids_hbm, tids_hbm, word_hbm, type_hbm, pos_hbm, gamma_hbm,
             beta_hbm, out_hbm,
             idx_v, tid_v, w_buf, t_buf, p_buf, o_buf, gam_v, bet_v,
             sem_w, sem_t):
    wid = lax.axis_index("s") * NC + lax.axis_index("c")
    pltpu.sync_copy(gamma_hbm, gam_v)
    pltpu.sync_copy(beta_hbm, bet_v)

    def chunk_body(c, carry):
        base = wid * TPW + c * CH
        pltpu.sync_copy(ids_hbm.at[pl.ds(base, CH)], idx_v)
        pltpu.sync_copy(tids_hbm.at[pl.ds(base, CH)], tid_v)
        cw = pltpu.async_copy(word_hbm.at[idx_v], w_buf, sem_w)
        ct = pltpu.async_copy(type_hbm.at[tid_v], t_buf, sem_t)
        pbase = lax.rem(base, S)
        pltpu.sync_copy(pos_hbm.at[pl.ds(pbase, CH)], p_buf)
        cw.wait()
        ct.wait()

        def tok_body(i, tc):
            zero = jnp.zeros((16,), jnp.float32)

            def ja(j, acc):
                s, s2 = acc
                sl = pl.ds(j * 16, 16)
                v = w_buf[i, sl] + p_buf[i, sl] + t_buf[i, sl]
                o_buf[i, sl] = v
                return s + v, s2 + v * v

            s, s2 = lax.fori_loop(0, DV, ja, (zero, zero))
            mean = _lanesum(s) * jnp.float32(1.0 / D)
            var = _lanesum(s2) * jnp.float32(1.0 / D) - mean * mean
            r = _rsqrt_vec(var + jnp.float32(EPS))

            def jc(j, cc):
                sl = pl.ds(j * 16, 16)
                v = o_buf[i, sl]
                o_buf[i, sl] = (v - mean) * r * gam_v[sl] + bet_v[sl]
                return cc

            lax.fori_loop(0, DV, jc, 0)
            return tc

        lax.fori_loop(0, CH, tok_body, 0)
        pltpu.sync_copy(o_buf, out_hbm.at[pl.ds(base, CH)])
        return carry

    lax.fori_loop(0, NCHUNK, chunk_body, 0)


def kernel(input_ids, token_type_ids, word_embeddings, token_type_embeddings,
           position_embeddings, ln_gamma, ln_beta):
    ids = input_ids.reshape(NTOK).astype(jnp.int32)
    tids = token_type_ids.reshape(NTOK).astype(jnp.int32)
    mesh = plsc.VectorSubcoreMesh(core_axis_name="c", subcore_axis_name="s")
    run = functools.partial(
        pl.kernel,
        mesh=mesh,
        out_type=jax.ShapeDtypeStruct((NTOK, D), jnp.float32),
        scratch_types=[
            pltpu.VMEM((CH,), jnp.int32),
            pltpu.VMEM((CH,), jnp.int32),
            pltpu.VMEM((CH, D), jnp.float32),
            pltpu.VMEM((CH, D), jnp.float32),
            pltpu.VMEM((CH, D), jnp.float32),
            pltpu.VMEM((CH, D), jnp.float32),
            pltpu.VMEM((D,), jnp.float32),
            pltpu.VMEM((D,), jnp.float32),
            pltpu.SemaphoreType.DMA,
            pltpu.SemaphoreType.DMA,
        ],
    )(_sc_body)
    out = run(ids, tids, word_embeddings, token_type_embeddings,
              position_embeddings, ln_gamma, ln_beta)
    return out.reshape(B, S, D)
